# Initial kernel scaffold; baseline (speedup 1.0000x reference)
import functools

import jax
import jax.numpy as jnp
from jax import lax
from jax.experimental import pallas as pl
from jax.experimental.pallas import tpu as pltpu
from jax.experimental.pallas import tpu_sc as plsc

D = 512
B = 16
NC, NS, L = 2, 16, 16
NW = NC * NS
CH = 64
DV = D // L


def _lane_extract(vec, lane):
    lanes = lax.iota(jnp.int32, L)
    return jnp.sum(jnp.where(lanes == lane, vec, jnp.zeros_like(vec)))


def _segment_sums_sc(objects, lengths):
    mesh = plsc.VectorSubcoreMesh(core_axis_name="c", subcore_axis_name="s")

    @functools.partial(
        pl.kernel,
        mesh=mesh,
        out_type=jax.ShapeDtypeStruct((NW, D), jnp.float32),
        scratch_types=[
            pltpu.VMEM((B,), jnp.int32),
            pltpu.VMEM((CH, D), jnp.float32),
            pltpu.VMEM((D,), jnp.float32),
        ],
    )
    def seg_sum(obj_hbm, len_hbm, out_hbm, lenv, buf, accv):
        wid = lax.axis_index("s") * NC + lax.axis_index("c")
        seg = lax.rem(wid, B)
        half = wid // B

        pltpu.sync_copy(len_hbm, lenv)
        lvec = lenv[...]
        csum = jnp.cumsum(lvec)
        startv = csum - lvec
        halfv = lvec // 2

        seg_start = _lane_extract(startv, seg)
        seg_len = _lane_extract(lvec, seg)
        half_len = _lane_extract(halfv, seg)
        start = seg_start + half * half_len
        count = jnp.where(half == 0, half_len, seg_len - half_len)
        nchunks = (count + CH - 1) // CH

        def chunk_body(i, acc):
            pltpu.sync_copy(obj_hbm.at[pl.ds(start + i * CH, CH)], buf)
            nrows = jnp.minimum(count - i * CH, CH)

            def row_body(r, a):
                return tuple(a[j] + buf[r, pl.ds(j * L, L)] for j in range(DV))

            return lax.fori_loop(0, nrows, row_body, acc)

        acc0 = tuple(jnp.zeros((L,), jnp.float32) for _ in range(DV))
        acc = lax.fori_loop(0, nchunks, chunk_body, acc0)
        for j in range(DV):
            accv[pl.ds(j * L, L)] = acc[j]
        pltpu.sync_copy(accv, out_hbm.at[wid])

    return seg_sum(objects, lengths)


def _head_tc(partials, lengths_f, W, b2, gamma2, beta2):
    def body(p_ref, len_ref, w_ref, b_ref, g_ref, be_ref, o_ref):
        sums = p_ref[0:B, :] + p_ref[B : 2 * B, :]
        lenf = len_ref[...]
        scale = jnp.where(lenf > 0, 1.0 / jnp.maximum(lenf, 1.0), 0.0)
        avg = sums * scale
        z = (
            lax.dot_general(
                avg,
                w_ref[...],
                (((1,), (1,)), ((), ())),
                preferred_element_type=jnp.float32,
            )
            + b_ref[...]
        )
        mean = jnp.mean(z, axis=0, keepdims=True)
        var = jnp.mean((z - mean) ** 2, axis=0, keepdims=True)
        o_ref[...] = g_ref[...] * (z - mean) * lax.rsqrt(var + 1e-5) + be_ref[...]

    return pl.pallas_call(
        body,
        out_shape=jax.ShapeDtypeStruct((B, D), jnp.float32),
    )(partials, lengths_f, W, b2, gamma2, beta2)


def kernel(objects_squares, lengths, W, b, gamma, beta):
    partials = _segment_sums_sc(objects_squares, lengths)
    return _head_tc(
        partials,
        lengths.astype(jnp.float32).reshape(B, 1),
        W,
        b.reshape(1, D),
        gamma.reshape(1, D),
        beta.reshape(1, D),
    )


# speedup vs baseline: 3.3124x; 3.3124x over previous
import functools

import jax
import jax.numpy as jnp
from jax import lax
from jax.experimental import pallas as pl
from jax.experimental.pallas import tpu as pltpu
from jax.experimental.pallas import tpu_sc as plsc

D = 512
B = 16
NC, NS, L = 2, 16, 16
NW = NC * NS
CH = 64
DV = D // L


def _lane_extract(vec, lane):
    lanes = lax.iota(jnp.int32, L)
    return jnp.sum(jnp.where(lanes == lane, vec, jnp.zeros_like(vec)))


def _segment_sums_sc(objects_flat, lengths):
    mesh = plsc.VectorSubcoreMesh(core_axis_name="c", subcore_axis_name="s")

    @functools.partial(
        pl.kernel,
        mesh=mesh,
        out_type=jax.ShapeDtypeStruct((NW * D,), jnp.float32),
        scratch_types=[
            pltpu.VMEM((B + L,), jnp.int32),
            pltpu.VMEM((CH * D,), jnp.float32),
            pltpu.VMEM((D,), jnp.float32),
        ],
    )
    def seg_sum(obj_hbm, len_hbm, out_hbm, lenv, buf, accv):
        wid = lax.axis_index("s") * NC + lax.axis_index("c")
        seg = lax.rem(wid, B)
        half = wid // B

        pltpu.sync_copy(len_hbm, lenv.at[pl.ds(0, B)])

        def scal(i):
            return lenv[pl.ds(i, L)][0]

        seg_start = lax.fori_loop(0, seg, lambda t, s: s + scal(t), jnp.int32(0))
        seg_len = scal(seg)
        half_len = seg_len // 2
        start = seg_start + half * half_len
        count = jnp.where(half == 0, half_len, seg_len - half_len)
        nchunks = (count + CH - 1) // CH

        def chunk_body(i, acc):
            pltpu.sync_copy(obj_hbm.at[pl.ds((start + i * CH) * D, CH * D)], buf)
            nrows = jnp.minimum(count - i * CH, CH)

            def row_body(r, a):
                base = r * D
                return tuple(a[j] + buf[pl.ds(base + j * L, L)] for j in range(DV))

            return lax.fori_loop(0, nrows, row_body, acc)

        acc0 = tuple(jnp.zeros((L,), jnp.float32) for _ in range(DV))
        acc = lax.fori_loop(0, nchunks, chunk_body, acc0)
        for j in range(DV):
            accv[pl.ds(j * L, L)] = acc[j]
        pltpu.sync_copy(accv, out_hbm.at[pl.ds(wid * D, D)])

    return seg_sum(objects_flat, lengths)


def _head_tc(partials, lengths_f, W, b2, gamma2, beta2):
    def body(p_ref, len_ref, w_ref, b_ref, g_ref, be_ref, o_ref):
        sums = p_ref[0:B, :] + p_ref[B : 2 * B, :]
        lenf = len_ref[...]
        scale = jnp.where(lenf > 0, 1.0 / jnp.maximum(lenf, 1.0), 0.0)
        avg = sums * scale
        z = (
            lax.dot_general(
                avg,
                w_ref[...],
                (((1,), (1,)), ((), ())),
                preferred_element_type=jnp.float32,
            )
            + b_ref[...]
        )
        mean = jnp.mean(z, axis=0, keepdims=True)
        var = jnp.mean((z - mean) ** 2, axis=0, keepdims=True)
        o_ref[...] = g_ref[...] * (z - mean) * lax.rsqrt(var + 1e-5) + be_ref[...]

    return pl.pallas_call(
        body,
        out_shape=jax.ShapeDtypeStruct((B, D), jnp.float32),
    )(partials, lengths_f, W, b2, gamma2, beta2)


def kernel(objects_squares, lengths, W, b, gamma, beta):
    partials_flat = _segment_sums_sc(objects_squares.reshape(-1), lengths)
    return _head_tc(
        partials_flat.reshape(NW, D),
        lengths.astype(jnp.float32).reshape(B, 1),
        W,
        b.reshape(1, D),
        gamma.reshape(1, D),
        beta.reshape(1, D),
    )


# speedup vs baseline: 6.5867x; 1.9885x over previous
import functools

import jax
import jax.numpy as jnp
from jax import lax
from jax.experimental import pallas as pl
from jax.experimental.pallas import tpu as pltpu
from jax.experimental.pallas import tpu_sc as plsc

D = 512
B = 16
NC, NS, L = 2, 16, 16
NW = NC * NS
CH = 64
DV = D // L


def _lane_extract(vec, lane):
    lanes = lax.iota(jnp.int32, L)
    return jnp.sum(jnp.where(lanes == lane, vec, jnp.zeros_like(vec)))


def _segment_sums_sc(objects, lengths):
    mesh = plsc.VectorSubcoreMesh(core_axis_name="c", subcore_axis_name="s")

    @functools.partial(
        pl.kernel,
        mesh=mesh,
        out_type=jax.ShapeDtypeStruct((NW * D,), jnp.float32),
        scratch_types=[
            pltpu.VMEM((B + L,), jnp.int32),
            pltpu.VMEM((CH + 8, D), jnp.float32),
            pltpu.VMEM((D,), jnp.float32),
        ],
    )
    def seg_sum(obj_hbm, len_hbm, out_hbm, lenv, buf, accv):
        wid = lax.axis_index("s") * NC + lax.axis_index("c")
        seg = lax.rem(wid, B)
        half = wid // B

        pltpu.sync_copy(len_hbm, lenv.at[pl.ds(0, B)])

        def scal(i):
            return lenv[pl.ds(i, L)][0]

        seg_start = lax.fori_loop(0, seg, lambda t, s: s + scal(t), jnp.int32(0))
        seg_len = scal(seg)
        half_len = seg_len // 2
        start = seg_start + half * half_len
        count = jnp.where(half == 0, half_len, seg_len - half_len)
        abase = (start // 8) * 8
        roff = start - abase
        nchunks = (count + CH - 1) // CH

        def chunk_body(i, acc):
            astart = pl.multiple_of(abase + i * CH, 8)
            pltpu.sync_copy(obj_hbm.at[pl.ds(astart, CH + 8)], buf)
            nrows = jnp.minimum(count - i * CH, CH)

            def row_body(r, a):
                row = roff + r
                return tuple(a[j] + buf[row, pl.ds(j * L, L)] for j in range(DV))

            return lax.fori_loop(0, nrows, row_body, acc)

        acc0 = tuple(jnp.zeros((L,), jnp.float32) for _ in range(DV))
        acc = lax.fori_loop(0, nchunks, chunk_body, acc0)
        for j in range(DV):
            accv[pl.ds(j * L, L)] = acc[j]
        pltpu.sync_copy(accv, out_hbm.at[pl.ds(wid * D, D)])

    return seg_sum(objects, lengths)


def _head_tc(partials, lengths_f, W, b2, gamma2, beta2):
    def body(p_ref, len_ref, w_ref, b_ref, g_ref, be_ref, o_ref):
        sums = p_ref[0:B, :] + p_ref[B : 2 * B, :]
        lenf = len_ref[...]
        scale = jnp.where(lenf > 0, 1.0 / jnp.maximum(lenf, 1.0), 0.0)
        avg = sums * scale
        z = (
            lax.dot_general(
                avg,
                w_ref[...],
                (((1,), (1,)), ((), ())),
                preferred_element_type=jnp.float32,
            )
            + b_ref[...]
        )
        mean = jnp.mean(z, axis=0, keepdims=True)
        var = jnp.mean((z - mean) ** 2, axis=0, keepdims=True)
        o_ref[...] = g_ref[...] * (z - mean) * lax.rsqrt(var + 1e-5) + be_ref[...]

    return pl.pallas_call(
        body,
        out_shape=jax.ShapeDtypeStruct((B, D), jnp.float32),
    )(partials, lengths_f, W, b2, gamma2, beta2)


def kernel(objects_squares, lengths, W, b, gamma, beta):
    partials_flat = _segment_sums_sc(objects_squares, lengths)
    return _head_tc(
        partials_flat.reshape(NW, D),
        lengths.astype(jnp.float32).reshape(B, 1),
        W,
        b.reshape(1, D),
        gamma.reshape(1, D),
        beta.reshape(1, D),
    )


# speedup vs baseline: 7.9265x; 1.2034x over previous
import functools

import jax
import jax.numpy as jnp
from jax import lax
from jax.experimental import pallas as pl
from jax.experimental.pallas import tpu as pltpu
from jax.experimental.pallas import tpu_sc as plsc

D = 512
B = 16
NC, NS, L = 2, 16, 16
NW = NC * NS
CH = 64
DV = D // L


def _lane_extract(vec, lane):
    lanes = lax.iota(jnp.int32, L)
    return jnp.sum(jnp.where(lanes == lane, vec, jnp.zeros_like(vec)))


def _segment_sums_sc(objects, lengths):
    mesh = plsc.VectorSubcoreMesh(core_axis_name="c", subcore_axis_name="s")

    @functools.partial(
        pl.kernel,
        mesh=mesh,
        out_type=jax.ShapeDtypeStruct((NW * D,), jnp.float32),
        scratch_types=[
            pltpu.VMEM((B + L,), jnp.int32),
            pltpu.VMEM((2, CH + 8, D), jnp.float32),
            pltpu.VMEM((D,), jnp.float32),
            pltpu.SemaphoreType.DMA((2,)),
        ],
    )
    def seg_sum(obj_hbm, len_hbm, out_hbm, lenv, buf, accv, sems):
        wid = lax.axis_index("s") * NC + lax.axis_index("c")
        seg = lax.rem(wid, B)
        half = wid // B

        pltpu.sync_copy(len_hbm, lenv.at[pl.ds(0, B)])

        def scal(i):
            return lenv[pl.ds(i, L)][0]

        seg_start = lax.fori_loop(0, seg, lambda t, s: s + scal(t), jnp.int32(0))
        seg_len = scal(seg)
        half_len = seg_len // 2
        start = seg_start + half * half_len
        count = jnp.where(half == 0, half_len, seg_len - half_len)
        abase = (start // 8) * 8
        roff = start - abase
        nchunks = (count + CH - 1) // CH

        def copy_desc(i, slot):
            astart = pl.multiple_of(abase + i * CH, 8)
            return pltpu.make_async_copy(
                obj_hbm.at[pl.ds(astart, CH + 8)], buf.at[slot], sems.at[slot]
            )

        @pl.when(nchunks > 0)
        def _():
            copy_desc(0, 0).start()

        def chunk_body(i, acc):
            slot = lax.rem(i, 2)
            copy_desc(i, slot).wait()

            @pl.when(i + 1 < nchunks)
            def _():
                copy_desc(i + 1, 1 - slot).start()

            nrows = jnp.minimum(count - i * CH, CH)

            def row_body(r, a):
                row = roff + r
                return tuple(a[j] + buf[slot, row, pl.ds(j * L, L)] for j in range(DV))

            return lax.fori_loop(0, nrows, row_body, acc)

        acc0 = tuple(jnp.zeros((L,), jnp.float32) for _ in range(DV))
        acc = lax.fori_loop(0, nchunks, chunk_body, acc0)
        for j in range(DV):
            accv[pl.ds(j * L, L)] = acc[j]
        pltpu.sync_copy(accv, out_hbm.at[pl.ds(wid * D, D)])

    return seg_sum(objects, lengths)


def _head_tc(partials, lengths_f, W, b2, gamma2, beta2):
    def body(p_ref, len_ref, w_ref, b_ref, g_ref, be_ref, o_ref):
        sums = p_ref[0:B, :] + p_ref[B : 2 * B, :]
        lenf = len_ref[...]
        scale = jnp.where(lenf > 0, 1.0 / jnp.maximum(lenf, 1.0), 0.0)
        avg = sums * scale
        z = (
            lax.dot_general(
                avg,
                w_ref[...],
                (((1,), (1,)), ((), ())),
                preferred_element_type=jnp.float32,
            )
            + b_ref[...]
        )
        mean = jnp.mean(z, axis=0, keepdims=True)
        var = jnp.mean((z - mean) ** 2, axis=0, keepdims=True)
        o_ref[...] = g_ref[...] * (z - mean) * lax.rsqrt(var + 1e-5) + be_ref[...]

    return pl.pallas_call(
        body,
        out_shape=jax.ShapeDtypeStruct((B, D), jnp.float32),
    )(partials, lengths_f, W, b2, gamma2, beta2)


def kernel(objects_squares, lengths, W, b, gamma, beta):
    partials_flat = _segment_sums_sc(objects_squares, lengths)
    return _head_tc(
        partials_flat.reshape(NW, D),
        lengths.astype(jnp.float32).reshape(B, 1),
        W,
        b.reshape(1, D),
        gamma.reshape(1, D),
        beta.reshape(1, D),
    )
